# Initial kernel scaffold; baseline (speedup 1.0000x reference)
import jax
import jax.numpy as jnp
from jax.experimental import pallas as pl


def kernel(x, node_idx, edge_idx, W_feat, b_feat, W_edge, b_edge, W_node, b_node):
    raise NotImplementedError("write your pallas kernel here")


# speedup vs baseline: 3.8743x; 3.8743x over previous
import functools

import jax
import jax.numpy as jnp
from jax import lax
from jax.experimental import pallas as pl
from jax.experimental.pallas import tpu as pltpu
from jax.experimental.pallas import tpu_sc as plsc

N, M, E, D = 10000, 2500, 320000, 128
NUM_LAYERS = 3

NCORES = 2
SUB = 16
NTILES = NCORES * SUB
CHUNK = 128
PER_TILE = -(-E // (NTILES * CHUNK)) * CHUNK
E_PAD = PER_TILE * NTILES
CH_PER_TILE = PER_TILE // CHUNK
M_ACC = 2560
N_ACC = 10112
F32 = jnp.float32

_MESH = plsc.VectorSubcoreMesh(core_axis_name="c", subcore_axis_name="s")


def _make_seg_sum(s_acc):
    stripe = s_acc // SUB

    @functools.partial(
        pl.kernel,
        out_type=jax.ShapeDtypeStruct((NCORES, s_acc, D), F32),
        mesh=_MESH,
        scratch_types=[
            pltpu.VMEM((CHUNK,), jnp.int32),
            pltpu.VMEM((CHUNK,), jnp.int32),
            pltpu.VMEM((CHUNK, D), F32),
            pltpu.VMEM_SHARED((s_acc, D), F32),
            pltpu.SemaphoreType.DMA,
        ],
    )
    def seg_sum(table, src_idx, dst_idx, zeros, out_acc,
                src_v, dst_v, rows_v, acc, sem):
        cid = lax.axis_index("c")
        sid = lax.axis_index("s")
        wid = sid * NCORES + cid
        r0 = sid * stripe
        pltpu.sync_copy(zeros.at[pl.ds(0, stripe)], acc.at[pl.ds(r0, stripe)])
        plsc.subcore_barrier()
        base0 = wid * PER_TILE

        def step(c, carry):
            b = base0 + c * CHUNK
            pltpu.sync_copy(src_idx.at[pl.ds(b, CHUNK)], src_v)
            pltpu.sync_copy(dst_idx.at[pl.ds(b, CHUNK)], dst_v)
            pltpu.async_copy(table.at[src_v], rows_v, sem).wait()
            pltpu.sync_copy(rows_v, acc.at[dst_v], add=True)
            return carry

        lax.fori_loop(0, CH_PER_TILE, step, 0)
        plsc.subcore_barrier()
        pltpu.sync_copy(acc.at[pl.ds(r0, stripe)],
                        out_acc.at[cid, pl.ds(r0, stripe)])

    return seg_sum


_seg_sum_edge = _make_seg_sum(M_ACC)
_seg_sum_node = _make_seg_sum(N_ACC)

_ESTRIPE = M_ACC // SUB
_NSTRIPE = N_ACC // SUB


@functools.partial(
    pl.kernel,
    out_type=(jax.ShapeDtypeStruct((NCORES, M_ACC, D), F32),
              jax.ShapeDtypeStruct((NCORES, N_ACC, D), F32)),
    mesh=_MESH,
    scratch_types=[
        pltpu.VMEM((CHUNK,), jnp.int32),
        pltpu.VMEM((CHUNK,), jnp.int32),
        pltpu.VMEM((CHUNK, D), F32),
        pltpu.VMEM_SHARED((M_ACC, D), F32),
        pltpu.VMEM_SHARED((N_ACC, D), F32),
    ],
)
def _seg_counts(ei, ni, ones_hbm, zeros, out_ce, out_cn,
                ev, nv, ones_v, ce, cn):
    cid = lax.axis_index("c")
    sid = lax.axis_index("s")
    wid = sid * NCORES + cid
    pltpu.sync_copy(zeros.at[pl.ds(0, _ESTRIPE)],
                    ce.at[pl.ds(sid * _ESTRIPE, _ESTRIPE)])
    pltpu.sync_copy(zeros.at[pl.ds(0, _NSTRIPE)],
                    cn.at[pl.ds(sid * _NSTRIPE, _NSTRIPE)])
    pltpu.sync_copy(ones_hbm, ones_v)
    plsc.subcore_barrier()
    base0 = wid * PER_TILE

    def step(c, carry):
        b = base0 + c * CHUNK
        pltpu.sync_copy(ei.at[pl.ds(b, CHUNK)], ev)
        pltpu.sync_copy(ni.at[pl.ds(b, CHUNK)], nv)
        pltpu.sync_copy(ones_v, ce.at[ev], add=True)
        pltpu.sync_copy(ones_v, cn.at[nv], add=True)
        return carry

    lax.fori_loop(0, CH_PER_TILE, step, 0)
    plsc.subcore_barrier()
    pltpu.sync_copy(ce.at[pl.ds(sid * _ESTRIPE, _ESTRIPE)],
                    out_ce.at[cid, pl.ds(sid * _ESTRIPE, _ESTRIPE)])
    pltpu.sync_copy(cn.at[pl.ds(sid * _NSTRIPE, _NSTRIPE)],
                    out_cn.at[cid, pl.ds(sid * _NSTRIPE, _NSTRIPE)])


def _dot(a, w):
    return lax.dot_general(a, w, (((1,), (0,)), ((), ())),
                           precision=lax.Precision.HIGHEST,
                           preferred_element_type=F32)


def _feat(x, w, b):
    def body(x_ref, w_ref, b_ref, o_ref):
        o_ref[...] = _dot(x_ref[...], w_ref[...]) + b_ref[...][None, :]

    return pl.pallas_call(
        body, out_shape=jax.ShapeDtypeStruct((N_ACC, D), F32))(x, w, b)


def _edge_mlp(acc, cnt, w, b):
    def body(a_ref, c_ref, w_ref, b_ref, o_ref):
        s = a_ref[0] + a_ref[1]
        c = jnp.maximum(c_ref[0] + c_ref[1], 1.0)
        o_ref[...] = jax.nn.gelu(_dot(s / c, w_ref[...]) + b_ref[...][None, :])

    return pl.pallas_call(
        body, out_shape=jax.ShapeDtypeStruct((M_ACC, D), F32))(acc, cnt, w, b)


_NBLK = N_ACC // 8


def _node_mlp(acc, cnt, w, b, h, hsum):
    def body(a_ref, c_ref, w_ref, b_ref, h_ref, s_ref, oh_ref, os_ref):
        s = a_ref[0] + a_ref[1]
        c = jnp.maximum(c_ref[0] + c_ref[1], 1.0)
        hn = jax.nn.gelu(_dot(s / c, w_ref[...]) + b_ref[...][None, :])
        hn = hn + h_ref[...]
        oh_ref[...] = hn
        os_ref[...] = s_ref[...] + hn

    row_spec = pl.BlockSpec((_NBLK, D), lambda i: (i, 0))
    acc_spec = pl.BlockSpec((NCORES, _NBLK, D), lambda i: (0, i, 0))
    return pl.pallas_call(
        body,
        grid=(N_ACC // _NBLK,),
        in_specs=[acc_spec, acc_spec,
                  pl.BlockSpec((D, D), lambda i: (0, 0)),
                  pl.BlockSpec((D,), lambda i: (0,)),
                  row_spec, row_spec],
        out_specs=(row_spec, row_spec),
        out_shape=(jax.ShapeDtypeStruct((N_ACC, D), F32),
                   jax.ShapeDtypeStruct((N_ACC, D), F32)),
    )(acc, cnt, w, b, h, hsum)


def _final_mean(hsum):
    def body(s_ref, o_ref):
        o_ref[...] = jnp.sum(s_ref[0:N, :], axis=0, keepdims=True) * (1.0 / N)

    return pl.pallas_call(
        body, out_shape=jax.ShapeDtypeStruct((1, D), F32))(hsum)


def kernel(x, node_idx, edge_idx, W_feat, b_feat, W_edge, b_edge,
           W_node, b_node):
    pad = E_PAD - E
    ni = jnp.concatenate(
        [node_idx.astype(jnp.int32), jnp.full((pad,), N, jnp.int32)])
    ei = jnp.concatenate(
        [edge_idx.astype(jnp.int32), jnp.full((pad,), M, jnp.int32)])
    xp = jnp.pad(x, ((0, N_ACC - N), (0, 0)))
    zeros = jnp.zeros((N_ACC // SUB, D), F32)
    ones_r = jnp.ones((CHUNK, D), F32)

    cnt_e, cnt_n = _seg_counts(ei, ni, ones_r, zeros)
    h = _feat(xp, W_feat, b_feat)
    hsum = jnp.zeros((N_ACC, D), F32)
    for _ in range(NUM_LAYERS):
        eacc = _seg_sum_edge(h, ni, ei, zeros)
        m = _edge_mlp(eacc, cnt_e, W_edge, b_edge)
        nacc = _seg_sum_node(m, ei, ni, zeros)
        h, hsum = _node_mlp(nacc, cnt_n, W_node, b_node, h, hsum)
    return _final_mean(hsum)


# speedup vs baseline: 4.4970x; 1.1607x over previous
import functools

import jax
import jax.numpy as jnp
from jax import lax
from jax.experimental import pallas as pl
from jax.experimental.pallas import tpu as pltpu
from jax.experimental.pallas import tpu_sc as plsc

N, M, E, D = 10000, 2500, 320000, 128
NUM_LAYERS = 3

NCORES = 2
SUB = 16
NTILES = NCORES * SUB
CHUNK = 128
LANES = 16
PER_TILE = -(-E // (NTILES * CHUNK)) * CHUNK
E_PAD = PER_TILE * NTILES
CH_PER_TILE = PER_TILE // CHUNK
M_ACC = 2560
N_ACC = 10112
F32 = jnp.float32

_MESH = plsc.VectorSubcoreMesh(core_axis_name="c", subcore_axis_name="s")


def _copy_idx(staged, c, dst):
    for j in range(CHUNK // LANES):
        dst[pl.ds(j * LANES, LANES)] = staged[c, 0, pl.ds(j * LANES, LANES)]


def _make_seg_sum(s_acc):
    stripe = s_acc // SUB

    @functools.partial(
        pl.kernel,
        out_type=jax.ShapeDtypeStruct((NCORES, s_acc, D), F32),
        mesh=_MESH,
        scratch_types=[
            pltpu.VMEM((CH_PER_TILE, 1, CHUNK), jnp.int32),
            pltpu.VMEM((CH_PER_TILE, 1, CHUNK), jnp.int32),
            pltpu.VMEM((CHUNK,), jnp.int32),
            pltpu.VMEM((CHUNK,), jnp.int32),
            pltpu.VMEM((CHUNK, D), F32),
            pltpu.VMEM_SHARED((s_acc, D), F32),
            pltpu.SemaphoreType.DMA,
        ],
    )
    def seg_sum(table, src_idx, dst_idx, zeros, out_acc,
                srcv, dstv, sbuf, dbuf, rows, acc, sem):
        cid = lax.axis_index("c")
        sid = lax.axis_index("s")
        wid = sid * NCORES + cid
        r0 = sid * stripe
        pltpu.sync_copy(zeros.at[pl.ds(0, stripe)], acc.at[pl.ds(r0, stripe)])
        row0 = wid * CH_PER_TILE
        pltpu.sync_copy(src_idx.at[pl.ds(row0, CH_PER_TILE)], srcv)
        pltpu.sync_copy(dst_idx.at[pl.ds(row0, CH_PER_TILE)], dstv)
        plsc.subcore_barrier()

        def step(c, carry):
            _copy_idx(srcv, c, sbuf)
            _copy_idx(dstv, c, dbuf)
            pltpu.async_copy(table.at[sbuf], rows, sem).wait()
            pltpu.sync_copy(rows, acc.at[dbuf], add=True)
            return carry

        lax.fori_loop(0, CH_PER_TILE, step, 0)
        plsc.subcore_barrier()
        pltpu.sync_copy(acc.at[pl.ds(r0, stripe)],
                        out_acc.at[cid, pl.ds(r0, stripe)])

    return seg_sum


_seg_sum_edge = _make_seg_sum(M_ACC)
_seg_sum_node = _make_seg_sum(N_ACC)


def _make_seg_count(s_acc):
    stripe = s_acc // SUB

    @functools.partial(
        pl.kernel,
        out_type=jax.ShapeDtypeStruct((NCORES, s_acc, D), F32),
        mesh=_MESH,
        scratch_types=[
            pltpu.VMEM((CH_PER_TILE, 1, CHUNK), jnp.int32),
            pltpu.VMEM((CHUNK,), jnp.int32),
            pltpu.VMEM((CHUNK, D), F32),
            pltpu.VMEM_SHARED((s_acc, D), F32),
        ],
    )
    def seg_count(dst_idx, ones_hbm, zeros, out_cnt, dstv, dbuf, ones_v, acc):
        cid = lax.axis_index("c")
        sid = lax.axis_index("s")
        wid = sid * NCORES + cid
        r0 = sid * stripe
        pltpu.sync_copy(zeros.at[pl.ds(0, stripe)], acc.at[pl.ds(r0, stripe)])
        pltpu.sync_copy(ones_hbm, ones_v)
        pltpu.sync_copy(dst_idx.at[pl.ds(wid * CH_PER_TILE, CH_PER_TILE)],
                        dstv)
        plsc.subcore_barrier()

        def step(c, carry):
            _copy_idx(dstv, c, dbuf)
            pltpu.sync_copy(ones_v, acc.at[dbuf], add=True)
            return carry

        lax.fori_loop(0, CH_PER_TILE, step, 0)
        plsc.subcore_barrier()
        pltpu.sync_copy(acc.at[pl.ds(r0, stripe)],
                        out_cnt.at[cid, pl.ds(r0, stripe)])

    return seg_count


_cnt_edge = _make_seg_count(M_ACC)
_cnt_node = _make_seg_count(N_ACC)


def _dot(a, w):
    return lax.dot_general(a, w, (((1,), (0,)), ((), ())),
                           precision=lax.Precision.HIGHEST,
                           preferred_element_type=F32)


def _feat(x, w, b):
    def body(x_ref, w_ref, b_ref, o_ref):
        o_ref[...] = _dot(x_ref[...], w_ref[...]) + b_ref[...][None, :]

    return pl.pallas_call(
        body, out_shape=jax.ShapeDtypeStruct((N_ACC, D), F32))(x, w, b)


def _edge_mlp(acc, cnt, w, b):
    def body(a_ref, c_ref, w_ref, b_ref, o_ref):
        s = a_ref[0] + a_ref[1]
        c = jnp.maximum(c_ref[0] + c_ref[1], 1.0)
        o_ref[...] = jax.nn.gelu(_dot(s / c, w_ref[...]) + b_ref[...][None, :])

    return pl.pallas_call(
        body, out_shape=jax.ShapeDtypeStruct((M_ACC, D), F32))(acc, cnt, w, b)


_NBLK = N_ACC // 8


def _node_mlp(acc, cnt, w, b, h, hsum):
    def body(a_ref, c_ref, w_ref, b_ref, h_ref, s_ref, oh_ref, os_ref):
        s = a_ref[0] + a_ref[1]
        c = jnp.maximum(c_ref[0] + c_ref[1], 1.0)
        hn = jax.nn.gelu(_dot(s / c, w_ref[...]) + b_ref[...][None, :])
        hn = hn + h_ref[...]
        oh_ref[...] = hn
        os_ref[...] = s_ref[...] + hn

    row_spec = pl.BlockSpec((_NBLK, D), lambda i: (i, 0))
    return pl.pallas_call(
        body,
        grid=(N_ACC // _NBLK,),
        in_specs=[pl.BlockSpec((NCORES, _NBLK, D), lambda i: (0, i, 0)),
                  pl.BlockSpec((NCORES, _NBLK, D), lambda i: (0, i, 0)),
                  pl.BlockSpec((D, D), lambda i: (0, 0)),
                  pl.BlockSpec((D,), lambda i: (0,)),
                  row_spec, row_spec],
        out_specs=(row_spec, row_spec),
        out_shape=(jax.ShapeDtypeStruct((N_ACC, D), F32),
                   jax.ShapeDtypeStruct((N_ACC, D), F32)),
    )(acc, cnt, w, b, h, hsum)


def _final_mean(hsum):
    def body(s_ref, o_ref):
        o_ref[...] = jnp.sum(s_ref[0:N, :], axis=0, keepdims=True) * (1.0 / N)

    return pl.pallas_call(
        body, out_shape=jax.ShapeDtypeStruct((1, D), F32))(hsum)


def kernel(x, node_idx, edge_idx, W_feat, b_feat, W_edge, b_edge,
           W_node, b_node):
    pad = E_PAD - E
    ni = jnp.concatenate(
        [node_idx.astype(jnp.int32), jnp.full((pad,), N, jnp.int32)])
    ei = jnp.concatenate(
        [edge_idx.astype(jnp.int32), jnp.full((pad,), M, jnp.int32)])
    ni = ni.reshape(E_PAD // CHUNK, 1, CHUNK)
    ei = ei.reshape(E_PAD // CHUNK, 1, CHUNK)
    xp = jnp.pad(x, ((0, N_ACC - N), (0, 0)))
    zeros = jnp.zeros((N_ACC // SUB, D), F32)
    ones_r = jnp.ones((CHUNK, D), F32)

    cnt_e = _cnt_edge(ei, ones_r, zeros)
    cnt_n = _cnt_node(ni, ones_r, zeros)
    h = _feat(xp, W_feat, b_feat)
    hsum = jnp.zeros((N_ACC, D), F32)
    for _ in range(NUM_LAYERS):
        eacc = _seg_sum_edge(h, ni, ei, zeros)
        m = _edge_mlp(eacc, cnt_e, W_edge, b_edge)
        nacc = _seg_sum_node(m, ei, ni, zeros)
        h, hsum = _node_mlp(nacc, cnt_n, W_node, b_node, h, hsum)
    return _final_mean(hsum)
